# Initial kernel scaffold; baseline (speedup 1.0000x reference)
import jax
import jax.numpy as jnp
from jax.experimental import pallas as pl


def kernel(predict, target, weight):
    raise NotImplementedError("write your pallas kernel here")


# speedup vs baseline: 11.2852x; 11.2852x over previous
import functools

import jax
import jax.numpy as jnp
from jax import lax
from jax.experimental import pallas as pl
from jax.experimental.pallas import tpu as pltpu
from jax.experimental.pallas import tpu_sc as plsc

N = 16 * 512 * 512
NORM = float(512 * 512 * 16)
MIN_KEPT = 100000
START = N - MIN_KEPT

NW = 32
PER = N // NW
C = 16384
NCH = PER // C

NB1 = 2048
NB2 = 4096
NB3 = 256

_INV = 1.0 / NORM


def _wid():
    return lax.axis_index("s") * 2 + lax.axis_index("c")


def _zero_hist(ref, nwords):
    zero16 = jnp.zeros((16,), ref.dtype)

    def body(k, _):
        ref[pl.ds(k * 16, 16)] = zero16
        return 0

    lax.fori_loop(0, nwords // 16, body, 0)


def _consts():
    lane = lax.iota(jnp.int32, 16)
    ones_i = jnp.ones((16,), jnp.int32)
    zero_i = jnp.zeros((16,), jnp.int32)
    zero_f = jnp.zeros((16,), jnp.float32)
    return lane, ones_i, zero_i, zero_f


def _build(interpret=False):
    _mesh = plsc.VectorSubcoreMesh(
        core_axis_name="c", subcore_axis_name="s",
        num_cores=2, num_subcores=16)

    @functools.partial(
        pl.kernel,
        out_type=jax.ShapeDtypeStruct((NW * NB1 * 16,), jnp.int32),
        mesh=_mesh,
        scratch_types=[
            pltpu.VMEM((NB1 * 16,), jnp.int32),
            pltpu.VMEM((C,), jnp.float32),
            pltpu.VMEM((C,), jnp.float32),
        ],
        compiler_params=pltpu.CompilerParams(needs_layout_passes=False),
        interpret=interpret,
    )
    def pass1(p_hbm, t_hbm, hist_hbm, hist_v, pbuf, tbuf):
        lane, ones_i, zero_i, zero_f = _consts()
        wid = _wid()
        _zero_hist(hist_v, NB1 * 16)

        def chunk(ci, _):
            base = wid * PER + ci * C
            pltpu.sync_copy(p_hbm.at[pl.ds(base, C)], pbuf)
            pltpu.sync_copy(t_hbm.at[pl.ds(base, C)], tbuf)

            def inner(j, _):
                pv = pbuf[pl.ds(j * 16, 16)]
                tv = tbuf[pl.ds(j * 16, 16)]
                d = pv - tv
                l = (d * d) * _INV
                bits = lax.bitcast_convert_type(l, jnp.int32)
                idx = ((bits >> 20) << 4) + lane
                plsc.addupdate_scatter(hist_v, [idx], ones_i)
                return 0

            lax.fori_loop(0, C // 16, inner, 0)
            return 0

        lax.fori_loop(0, NCH, chunk, 0)
        pltpu.sync_copy(hist_v, hist_hbm.at[pl.ds(wid * NB1 * 16, NB1 * 16)])

    @functools.partial(
        pl.kernel,
        out_type=(
            jax.ShapeDtypeStruct((NW * NB2 * 16,), jnp.int32),
            jax.ShapeDtypeStruct((NW * 16,), jnp.int32),
            jax.ShapeDtypeStruct((NW * 16,), jnp.float32),
        ),
        mesh=_mesh,
        scratch_types=[
            pltpu.VMEM((NB2 * 16,), jnp.int32),
            pltpu.VMEM((C,), jnp.float32),
            pltpu.VMEM((C,), jnp.float32),
            pltpu.VMEM((C,), jnp.float32),
            pltpu.VMEM((16,), jnp.int32),
            pltpu.VMEM((16,), jnp.int32),
            pltpu.VMEM((16,), jnp.float32),
        ],
        compiler_params=pltpu.CompilerParams(needs_layout_passes=False),
        interpret=interpret,
    )
    def pass2(p_hbm, t_hbm, w_hbm, d1_hbm, hist_hbm, cnt_hbm, ws_hbm,
              hist_v, pbuf, tbuf, wbuf, d1_v, cnt_v, ws_v):
        lane, ones_i, zero_i, zero_f = _consts()
        wid = _wid()
        _zero_hist(hist_v, NB2 * 16)
        pltpu.sync_copy(d1_hbm, d1_v)
        d1 = d1_v[...]

        def chunk(ci, carry):
            base = wid * PER + ci * C
            pltpu.sync_copy(p_hbm.at[pl.ds(base, C)], pbuf)
            pltpu.sync_copy(t_hbm.at[pl.ds(base, C)], tbuf)
            pltpu.sync_copy(w_hbm.at[pl.ds(base, C)], wbuf)

            def inner(j, c2):
                cnt, ws = c2
                pv = pbuf[pl.ds(j * 16, 16)]
                tv = tbuf[pl.ds(j * 16, 16)]
                wv = wbuf[pl.ds(j * 16, 16)]
                d = pv - tv
                d2 = d * d
                wl = (wv * d2) * _INV
                bits = lax.bitcast_convert_type(d2 * _INV, jnp.int32)
                hi = bits >> 20
                mid = (bits >> 8) & 0xFFF
                idx = (mid << 4) + lane
                plsc.addupdate_scatter(hist_v, [idx], ones_i, mask=hi == d1)
                m_ab = hi > d1
                cnt = cnt + jnp.where(m_ab, ones_i, zero_i)
                ws = ws + jnp.where(m_ab, wl, zero_f)
                return (cnt, ws)

            return lax.fori_loop(0, C // 16, inner, carry)

        cnt, ws = lax.fori_loop(0, NCH, chunk, (zero_i, zero_f))
        cnt_v[...] = cnt
        ws_v[...] = ws
        pltpu.sync_copy(hist_v, hist_hbm.at[pl.ds(wid * NB2 * 16, NB2 * 16)])
        pltpu.sync_copy(cnt_v, cnt_hbm.at[pl.ds(wid * 16, 16)])
        pltpu.sync_copy(ws_v, ws_hbm.at[pl.ds(wid * 16, 16)])

    @functools.partial(
        pl.kernel,
        out_type=(
            jax.ShapeDtypeStruct((NW * NB3 * 16,), jnp.int32),
            jax.ShapeDtypeStruct((NW * NB3 * 16,), jnp.float32),
            jax.ShapeDtypeStruct((NW * 16,), jnp.int32),
            jax.ShapeDtypeStruct((NW * 16,), jnp.float32),
        ),
        mesh=_mesh,
        scratch_types=[
            pltpu.VMEM((NB3 * 16,), jnp.int32),
            pltpu.VMEM((NB3 * 16,), jnp.float32),
            pltpu.VMEM((C,), jnp.float32),
            pltpu.VMEM((C,), jnp.float32),
            pltpu.VMEM((C,), jnp.float32),
            pltpu.VMEM((16,), jnp.int32),
            pltpu.VMEM((16,), jnp.int32),
            pltpu.VMEM((16,), jnp.int32),
            pltpu.VMEM((16,), jnp.float32),
        ],
        compiler_params=pltpu.CompilerParams(needs_layout_passes=False),
        interpret=interpret,
    )
    def pass3(p_hbm, t_hbm, w_hbm, t24_hbm, hiend_hbm,
              hist_hbm, whist_hbm, cnt_hbm, ws_hbm,
              hist_v, whist_v, pbuf, tbuf, wbuf, t24_v, hiend_v, cnt_v, ws_v):
        lane, ones_i, zero_i, zero_f = _consts()
        wid = _wid()
        _zero_hist(hist_v, NB3 * 16)
        _zero_hist(whist_v, NB3 * 16)
        pltpu.sync_copy(t24_hbm, t24_v)
        pltpu.sync_copy(hiend_hbm, hiend_v)
        t24 = t24_v[...]
        hiend = hiend_v[...]

        def chunk(ci, carry):
            base = wid * PER + ci * C
            pltpu.sync_copy(p_hbm.at[pl.ds(base, C)], pbuf)
            pltpu.sync_copy(t_hbm.at[pl.ds(base, C)], tbuf)
            pltpu.sync_copy(w_hbm.at[pl.ds(base, C)], wbuf)

            def inner(j, c2):
                cnt, ws = c2
                pv = pbuf[pl.ds(j * 16, 16)]
                tv = tbuf[pl.ds(j * 16, 16)]
                wv = wbuf[pl.ds(j * 16, 16)]
                d = pv - tv
                d2 = d * d
                wl = (wv * d2) * _INV
                bits = lax.bitcast_convert_type(d2 * _INV, jnp.int32)
                p24 = bits >> 8
                m_in = p24 == t24
                idx = ((bits & 0xFF) << 4) + lane
                plsc.addupdate_scatter(hist_v, [idx], ones_i, mask=m_in)
                plsc.addupdate_scatter(whist_v, [idx], wl, mask=m_in)
                m_ab = (p24 > t24) & (p24 <= hiend)
                cnt = cnt + jnp.where(m_ab, ones_i, zero_i)
                ws = ws + jnp.where(m_ab, wl, zero_f)
                return (cnt, ws)

            return lax.fori_loop(0, C // 16, inner, carry)

        cnt, ws = lax.fori_loop(0, NCH, chunk, (zero_i, zero_f))
        cnt_v[...] = cnt
        ws_v[...] = ws
        pltpu.sync_copy(hist_v, hist_hbm.at[pl.ds(wid * NB3 * 16, NB3 * 16)])
        pltpu.sync_copy(whist_v, whist_hbm.at[pl.ds(wid * NB3 * 16, NB3 * 16)])
        pltpu.sync_copy(cnt_v, cnt_hbm.at[pl.ds(wid * 16, 16)])
        pltpu.sync_copy(ws_v, ws_hbm.at[pl.ds(wid * 16, 16)])

    return pass1, pass2, pass3


_PASSES = None


def _get_passes():
    global _PASSES
    if _PASSES is None:
        _PASSES = _build()
    return _PASSES


def kernel(predict, target, weight):
    _pass1, _pass2, _pass3 = _get_passes()
    p = predict.reshape(-1)
    t = target.reshape(-1)
    w = weight.reshape(-1)

    h1 = _pass1(p, t).reshape(NW, NB1, 16).sum(axis=(0, 2))
    cum1 = jnp.cumsum(h1)
    d1 = jnp.sum((cum1 <= START).astype(jnp.int32))
    r1 = START - (cum1[d1] - h1[d1])

    d1v = jnp.full((16,), d1, jnp.int32)
    h2raw, cnt2, ws2 = _pass2(p, t, w, d1v)
    h2 = h2raw.reshape(NW, NB2, 16).sum(axis=(0, 2))
    cum2 = jnp.cumsum(h2)
    d2 = jnp.sum((cum2 <= r1).astype(jnp.int32))
    r2 = r1 - (cum2[d2] - h2[d2])

    t24 = d1 * NB2 + d2
    t24v = jnp.full((16,), t24, jnp.int32)
    hiendv = jnp.full((16,), d1 * NB2 + (NB2 - 1), jnp.int32)
    h3raw, wh3raw, cnt3, ws3 = _pass3(p, t, w, t24v, hiendv)
    h3 = h3raw.reshape(NW, NB3, 16).sum(axis=(0, 2))
    wh3 = wh3raw.reshape(NW, NB3, 16).sum(axis=(0, 2))
    cum3 = jnp.cumsum(h3)
    d3 = jnp.sum((cum3 <= r2).astype(jnp.int32))

    bins = jnp.arange(NB3)
    cnt_in = jnp.sum(jnp.where(bins > d3, h3, 0))
    ws_in = jnp.sum(jnp.where(bins > d3, wh3, 0.0))

    sel_cnt = cnt2.sum() + cnt3.sum() + cnt_in
    sel_sum = ws2.sum() + ws3.sum() + ws_in
    return sel_sum / jnp.maximum(sel_cnt, 1).astype(jnp.float32)


# speedup vs baseline: 12.7942x; 1.1337x over previous
import functools

import jax
import jax.numpy as jnp
from jax import lax
from jax.experimental import pallas as pl
from jax.experimental.pallas import tpu as pltpu
from jax.experimental.pallas import tpu_sc as plsc

N = 16 * 512 * 512
NORM = float(512 * 512 * 16)
MIN_KEPT = 100000
START = N - MIN_KEPT

NW = 32
PER = N // NW
C = 8192
NCH = PER // C

NB1 = 2048
NB2 = 4096
NB3 = 256

_INV = 1.0 / NORM


def _wid():
    return lax.axis_index("s") * 2 + lax.axis_index("c")


def _zero_hist(ref, nwords):
    zero16 = jnp.zeros((16,), ref.dtype)

    def body(k, _):
        ref[pl.ds(k * 16, 16)] = zero16
        return 0

    lax.fori_loop(0, nwords // 16, body, 0)


def _stream(wid, hbm_refs, bufs0, bufs1, sem0, sem1, compute, init):
    def start(ci, bufs, sem):
        base = wid * PER + ci * C
        return [pltpu.async_copy(a.at[pl.ds(base, C)], b, sem)
                for a, b in zip(hbm_refs, bufs)]

    slots = (bufs0, bufs1)
    sems = (sem0, sem1)
    carry = init
    handles = {0: start(0, slots[0], sems[0])}
    for ci in range(NCH):
        if ci + 1 < NCH:
            s = (ci + 1) % 2
            handles[ci + 1] = start(ci + 1, slots[s], sems[s])
        for h in handles.pop(ci):
            h.wait()
        carry = compute(slots[ci % 2], carry)
    return carry


def _consts():
    lane = lax.iota(jnp.int32, 16)
    ones_i = jnp.ones((16,), jnp.int32)
    zero_i = jnp.zeros((16,), jnp.int32)
    zero_f = jnp.zeros((16,), jnp.float32)
    return lane, ones_i, zero_i, zero_f


def _build(interpret=False):
    _mesh = plsc.VectorSubcoreMesh(
        core_axis_name="c", subcore_axis_name="s",
        num_cores=2, num_subcores=16)

    @functools.partial(
        pl.kernel,
        out_type=jax.ShapeDtypeStruct((NW * NB1 * 16,), jnp.int32),
        mesh=_mesh,
        scratch_types=[
            pltpu.VMEM((NB1 * 16,), jnp.int32),
            pltpu.VMEM((C,), jnp.float32),
            pltpu.VMEM((C,), jnp.float32),
            pltpu.VMEM((C,), jnp.float32),
            pltpu.VMEM((C,), jnp.float32),
            pltpu.SemaphoreType.DMA,
            pltpu.SemaphoreType.DMA,
        ],
        compiler_params=pltpu.CompilerParams(needs_layout_passes=False),
        interpret=interpret,
    )
    def pass1(p_hbm, t_hbm, hist_hbm,
              hist_v, pb0, tb0, pb1, tb1, sem0, sem1):
        lane, ones_i, zero_i, zero_f = _consts()
        wid = _wid()
        _zero_hist(hist_v, NB1 * 16)

        def compute(bufs, carry):
            pbuf, tbuf = bufs

            def inner(j, _):
                pv = pbuf[pl.ds(j * 16, 16)]
                tv = tbuf[pl.ds(j * 16, 16)]
                d = pv - tv
                l = (d * d) * _INV
                bits = lax.bitcast_convert_type(l, jnp.int32)
                idx = ((bits >> 20) << 4) + lane
                plsc.addupdate_scatter(hist_v, [idx], ones_i)
                return 0

            lax.fori_loop(0, C // 16, inner, 0)
            return carry

        _stream(wid, [p_hbm, t_hbm], [pb0, tb0], [pb1, tb1],
                sem0, sem1, compute, 0)
        pltpu.sync_copy(hist_v, hist_hbm.at[pl.ds(wid * NB1 * 16, NB1 * 16)])

    @functools.partial(
        pl.kernel,
        out_type=(
            jax.ShapeDtypeStruct((NW * NB2 * 16,), jnp.int32),
            jax.ShapeDtypeStruct((NW * 16,), jnp.int32),
            jax.ShapeDtypeStruct((NW * 16,), jnp.float32),
        ),
        mesh=_mesh,
        scratch_types=[
            pltpu.VMEM((NB2 * 16,), jnp.int32),
            pltpu.VMEM((C,), jnp.float32),
            pltpu.VMEM((C,), jnp.float32),
            pltpu.VMEM((C,), jnp.float32),
            pltpu.VMEM((C,), jnp.float32),
            pltpu.VMEM((C,), jnp.float32),
            pltpu.VMEM((C,), jnp.float32),
            pltpu.VMEM((16,), jnp.int32),
            pltpu.VMEM((16,), jnp.int32),
            pltpu.VMEM((16,), jnp.float32),
            pltpu.SemaphoreType.DMA,
            pltpu.SemaphoreType.DMA,
        ],
        compiler_params=pltpu.CompilerParams(needs_layout_passes=False),
        interpret=interpret,
    )
    def pass2(p_hbm, t_hbm, w_hbm, d1_hbm, hist_hbm, cnt_hbm, ws_hbm,
              hist_v, pb0, tb0, wb0, pb1, tb1, wb1, d1_v, cnt_v, ws_v,
              sem0, sem1):
        lane, ones_i, zero_i, zero_f = _consts()
        wid = _wid()
        _zero_hist(hist_v, NB2 * 16)
        pltpu.sync_copy(d1_hbm, d1_v)
        d1 = d1_v[...]

        def compute(bufs, carry):
            pbuf, tbuf, wbuf = bufs

            def inner(j, c2):
                cnt, ws = c2
                pv = pbuf[pl.ds(j * 16, 16)]
                tv = tbuf[pl.ds(j * 16, 16)]
                wv = wbuf[pl.ds(j * 16, 16)]
                d = pv - tv
                d2 = d * d
                wl = (wv * d2) * _INV
                bits = lax.bitcast_convert_type(d2 * _INV, jnp.int32)
                hi = bits >> 20
                mid = (bits >> 8) & 0xFFF
                idx = (mid << 4) + lane
                plsc.addupdate_scatter(hist_v, [idx], ones_i, mask=hi == d1)
                m_ab = hi > d1
                cnt = cnt + jnp.where(m_ab, ones_i, zero_i)
                ws = ws + jnp.where(m_ab, wl, zero_f)
                return (cnt, ws)

            return lax.fori_loop(0, C // 16, inner, carry)

        cnt, ws = _stream(wid, [p_hbm, t_hbm, w_hbm], [pb0, tb0, wb0],
                          [pb1, tb1, wb1], sem0, sem1, compute,
                          (zero_i, zero_f))
        cnt_v[...] = cnt
        ws_v[...] = ws
        pltpu.sync_copy(hist_v, hist_hbm.at[pl.ds(wid * NB2 * 16, NB2 * 16)])
        pltpu.sync_copy(cnt_v, cnt_hbm.at[pl.ds(wid * 16, 16)])
        pltpu.sync_copy(ws_v, ws_hbm.at[pl.ds(wid * 16, 16)])

    @functools.partial(
        pl.kernel,
        out_type=(
            jax.ShapeDtypeStruct((NW * NB3 * 16,), jnp.int32),
            jax.ShapeDtypeStruct((NW * NB3 * 16,), jnp.float32),
            jax.ShapeDtypeStruct((NW * 16,), jnp.int32),
            jax.ShapeDtypeStruct((NW * 16,), jnp.float32),
        ),
        mesh=_mesh,
        scratch_types=[
            pltpu.VMEM((NB3 * 16,), jnp.int32),
            pltpu.VMEM((NB3 * 16,), jnp.float32),
            pltpu.VMEM((C,), jnp.float32),
            pltpu.VMEM((C,), jnp.float32),
            pltpu.VMEM((C,), jnp.float32),
            pltpu.VMEM((C,), jnp.float32),
            pltpu.VMEM((C,), jnp.float32),
            pltpu.VMEM((C,), jnp.float32),
            pltpu.VMEM((16,), jnp.int32),
            pltpu.VMEM((16,), jnp.int32),
            pltpu.VMEM((16,), jnp.int32),
            pltpu.VMEM((16,), jnp.float32),
            pltpu.SemaphoreType.DMA,
            pltpu.SemaphoreType.DMA,
        ],
        compiler_params=pltpu.CompilerParams(needs_layout_passes=False),
        interpret=interpret,
    )
    def pass3(p_hbm, t_hbm, w_hbm, t24_hbm, hiend_hbm,
              hist_hbm, whist_hbm, cnt_hbm, ws_hbm,
              hist_v, whist_v, pb0, tb0, wb0, pb1, tb1, wb1,
              t24_v, hiend_v, cnt_v, ws_v, sem0, sem1):
        lane, ones_i, zero_i, zero_f = _consts()
        wid = _wid()
        _zero_hist(hist_v, NB3 * 16)
        _zero_hist(whist_v, NB3 * 16)
        pltpu.sync_copy(t24_hbm, t24_v)
        pltpu.sync_copy(hiend_hbm, hiend_v)
        t24 = t24_v[...]
        hiend = hiend_v[...]

        def compute(bufs, carry):
            pbuf, tbuf, wbuf = bufs

            def inner(j, c2):
                cnt, ws = c2
                pv = pbuf[pl.ds(j * 16, 16)]
                tv = tbuf[pl.ds(j * 16, 16)]
                wv = wbuf[pl.ds(j * 16, 16)]
                d = pv - tv
                d2 = d * d
                wl = (wv * d2) * _INV
                bits = lax.bitcast_convert_type(d2 * _INV, jnp.int32)
                p24 = bits >> 8
                m_in = p24 == t24
                idx = ((bits & 0xFF) << 4) + lane
                plsc.addupdate_scatter(hist_v, [idx], ones_i, mask=m_in)
                plsc.addupdate_scatter(whist_v, [idx], wl, mask=m_in)
                m_ab = (p24 > t24) & (p24 <= hiend)
                cnt = cnt + jnp.where(m_ab, ones_i, zero_i)
                ws = ws + jnp.where(m_ab, wl, zero_f)
                return (cnt, ws)

            return lax.fori_loop(0, C // 16, inner, carry)

        cnt, ws = _stream(wid, [p_hbm, t_hbm, w_hbm], [pb0, tb0, wb0],
                          [pb1, tb1, wb1], sem0, sem1, compute,
                          (zero_i, zero_f))
        cnt_v[...] = cnt
        ws_v[...] = ws
        pltpu.sync_copy(hist_v, hist_hbm.at[pl.ds(wid * NB3 * 16, NB3 * 16)])
        pltpu.sync_copy(whist_v, whist_hbm.at[pl.ds(wid * NB3 * 16, NB3 * 16)])
        pltpu.sync_copy(cnt_v, cnt_hbm.at[pl.ds(wid * 16, 16)])
        pltpu.sync_copy(ws_v, ws_hbm.at[pl.ds(wid * 16, 16)])

    return pass1, pass2, pass3


_PASSES = None


def _get_passes():
    global _PASSES
    if _PASSES is None:
        _PASSES = _build()
    return _PASSES


def kernel(predict, target, weight):
    _pass1, _pass2, _pass3 = _get_passes()
    p = predict.reshape(-1)
    t = target.reshape(-1)
    w = weight.reshape(-1)

    h1 = _pass1(p, t).reshape(NW, NB1, 16).sum(axis=(0, 2))
    cum1 = jnp.cumsum(h1)
    d1 = jnp.sum((cum1 <= START).astype(jnp.int32))
    r1 = START - (cum1[d1] - h1[d1])

    d1v = jnp.full((16,), d1, jnp.int32)
    h2raw, cnt2, ws2 = _pass2(p, t, w, d1v)
    h2 = h2raw.reshape(NW, NB2, 16).sum(axis=(0, 2))
    cum2 = jnp.cumsum(h2)
    d2 = jnp.sum((cum2 <= r1).astype(jnp.int32))
    r2 = r1 - (cum2[d2] - h2[d2])

    t24 = d1 * NB2 + d2
    t24v = jnp.full((16,), t24, jnp.int32)
    hiendv = jnp.full((16,), d1 * NB2 + (NB2 - 1), jnp.int32)
    h3raw, wh3raw, cnt3, ws3 = _pass3(p, t, w, t24v, hiendv)
    h3 = h3raw.reshape(NW, NB3, 16).sum(axis=(0, 2))
    wh3 = wh3raw.reshape(NW, NB3, 16).sum(axis=(0, 2))
    cum3 = jnp.cumsum(h3)
    d3 = jnp.sum((cum3 <= r2).astype(jnp.int32))

    bins = jnp.arange(NB3)
    cnt_in = jnp.sum(jnp.where(bins > d3, h3, 0))
    ws_in = jnp.sum(jnp.where(bins > d3, wh3, 0.0))

    sel_cnt = cnt2.sum() + cnt3.sum() + cnt_in
    sel_sum = ws2.sum() + ws3.sum() + ws_in
    return sel_sum / jnp.maximum(sel_cnt, 1).astype(jnp.float32)
